# baseline (speedup 1.0000x reference)
import functools

import jax
import jax.numpy as jnp
from jax import lax
from jax.experimental import pallas as pl
from jax.experimental.pallas import tpu as pltpu
from jax.experimental.pallas import tpu_sc as plsc

_BLK = 4096
_MP = 1024
_NC = 2
_NS = 16
_CHUNK = 128


def _phase1_body(m, q_ref, k_ref, wq_ref, g_ref, cm_ref, cms):
    i = pl.program_id(0)
    qb = q_ref[...]
    ss = jnp.sum(qb * qb, axis=1, keepdims=True)
    qn = qb * jax.lax.rsqrt(ss + 1e-24)
    scT = jax.lax.dot_general(
        k_ref[...], qn.astype(jnp.bfloat16),
        (((1,), (1,)), ((), ())), preferred_element_type=jnp.float32)
    ri = jax.lax.broadcasted_iota(jnp.int32, (_MP, 1), 0)
    offv = jnp.where(ri < m, jnp.float32(32.0), -jnp.inf)
    invv = jnp.where(ri < m, (_MP - 1) - ri, 0)
    sc2 = scT + offv
    kb = jax.lax.bitcast_convert_type(sc2, jnp.int32)
    keyf = jax.lax.bitcast_convert_type(
        (kb & jnp.int32(-1024)) | invv, jnp.float32)
    gk = jnp.max(keyf, axis=0, keepdims=True)
    colp = jnp.max(sc2, axis=1, keepdims=True)
    gb = jax.lax.bitcast_convert_type(gk, jnp.int32)
    g = (_MP - 1) - (gb & 1023)
    s2 = jax.lax.bitcast_convert_type(gb & jnp.int32(-1024), jnp.float32)
    es = jnp.exp(s2)
    esT = jax.lax.transpose(jnp.broadcast_to(es, (128, _BLK)), (1, 0))
    wq_ref[...] = qn * esT
    g_ref[...] = g.reshape(1, 1, _BLK)

    @pl.when(i == 0)
    def _():
        cms[...] = colp

    @pl.when(i > 0)
    def _():
        cms[...] = jnp.maximum(cms[...], colp)

    @pl.when(i == pl.num_programs(0) - 1)
    def _():
        cm_ref[...] = cms[...]


def _scatter_body(n_tok, wq_hbm, g2_hbm, z_hbm, out_hbm,
                  idx_v, bufs, gsems, acc_sh):
    c = lax.axis_index("c")
    s = lax.axis_index("s")
    wid = s * _NC + c
    per_w = n_tok // (_NC * _NS)
    nch = per_w // _CHUNK

    @pl.when(s == 0)
    def _():
        pltpu.sync_copy(z_hbm, acc_sh)

    pltpu.sync_copy(g2_hbm.at[pl.ds(wid * nch, nch)], idx_v)

    base = wid * per_w
    gh = [pltpu.async_copy(
        wq_hbm.at[pl.ds(base + ch * _CHUNK, _CHUNK)], bufs[ch], gsems[ch])
        for ch in range(nch)]
    plsc.subcore_barrier()
    for ch in range(nch):
        gh[ch].wait()
        pltpu.sync_copy(bufs[ch], acc_sh.at[idx_v.at[ch]], add=True)

    plsc.subcore_barrier()
    rows_per_tile = _MP // _NS
    pltpu.sync_copy(acc_sh.at[pl.ds(s * rows_per_tile, rows_per_tile)],
                    out_hbm.at[pl.ds(c * _MP + s * rows_per_tile, rows_per_tile)])


def _finalize_body(m, np_, k_ref, *refs):
    acc_refs = refs[:np_]
    cm_refs = refs[np_:2 * np_]
    out_ref = refs[2 * np_]
    upd = acc_refs[0][0:_MP, :] + acc_refs[0][_MP:2 * _MP, :]
    cm = cm_refs[0][...]
    for j in range(1, np_):
        upd = upd + acc_refs[j][0:_MP, :] + acc_refs[j][_MP:2 * _MP, :]
        cm = jnp.maximum(cm, cm_refs[j][...])
    x = 1e-5 * (upd[0:m, :] * jnp.exp(-cm[0:m, :])) + k_ref[...]
    nrm = jnp.sqrt(jnp.sum(x * x, axis=1, keepdims=True))
    out_ref[...] = x / jnp.maximum(nrm, 1e-12)


_NP = 2


def kernel(query, keys):
    bs, t, d = query.shape
    m = keys.shape[0]
    n_tok = bs * t
    nb = n_tok // _BLK
    q2 = query.reshape(n_tok, d)
    kp = jnp.pad(keys, ((0, _MP - m), (0, 0))).astype(jnp.bfloat16)

    npb = nb // _NP
    ck_tok = n_tok // _NP
    nch = ck_tok // (_NC * _NS) // _CHUNK
    mesh = plsc.VectorSubcoreMesh(
        core_axis_name="c", subcore_axis_name="s",
        num_cores=_NC, num_subcores=_NS)
    zeros = jnp.zeros((_MP, d), jnp.float32)

    accs, cms = [], []
    for j in range(_NP):
        wq_j, g_j, cm_j = pl.pallas_call(
            functools.partial(_phase1_body, m),
            grid=(npb,),
            in_specs=[
                pl.BlockSpec((_BLK, d), lambda i, j=j: (j * npb + i, 0)),
                pl.BlockSpec((_MP, d), lambda i: (0, 0)),
            ],
            out_specs=[
                pl.BlockSpec((_BLK, d), lambda i: (i, 0)),
                pl.BlockSpec((1, 1, _BLK), lambda i: (i, 0, 0)),
                pl.BlockSpec((_MP, 1), lambda i: (0, 0)),
            ],
            out_shape=[
                jax.ShapeDtypeStruct((ck_tok, d), jnp.float32),
                jax.ShapeDtypeStruct((npb, 1, _BLK), jnp.int32),
                jax.ShapeDtypeStruct((_MP, 1), jnp.float32),
            ],
            scratch_shapes=[pltpu.VMEM((_MP, 1), jnp.float32)],
        )(q2, kp)

        acc_j = pl.kernel(
            functools.partial(_scatter_body, ck_tok),
            out_type=jax.ShapeDtypeStruct((_NC * _MP, d), jnp.float32),
            mesh=mesh,
            scratch_types=[
                pltpu.VMEM((nch, _CHUNK), jnp.int32),
                tuple(pltpu.VMEM((_CHUNK, d), jnp.float32) for _ in range(nch)),
                tuple(pltpu.SemaphoreType.DMA for _ in range(nch)),
                pltpu.VMEM_SHARED((_MP, d), jnp.float32),
            ],
        )(wq_j, g_j.reshape(ck_tok // _CHUNK, _CHUNK), zeros)
        accs.append(acc_j)
        cms.append(cm_j)

    out = pl.pallas_call(
        functools.partial(_finalize_body, m, _NP),
        in_specs=[pl.BlockSpec((m, d), lambda: (0, 0))]
        + [pl.BlockSpec((_NC * _MP, d), lambda: (0, 0))] * _NP
        + [pl.BlockSpec((_MP, 1), lambda: (0, 0))] * _NP,
        out_specs=pl.BlockSpec((m, d), lambda: (0, 0)),
        out_shape=jax.ShapeDtypeStruct((m, d), jnp.float32),
    )(keys, *accs, *cms)
    return out


# speedup vs baseline: 1.0286x; 1.0286x over previous
import functools

import jax
import jax.numpy as jnp
from jax import lax
from jax.experimental import pallas as pl
from jax.experimental.pallas import tpu as pltpu
from jax.experimental.pallas import tpu_sc as plsc

_BLK = 4096
_MP = 1024
_NC = 2
_NS = 16
_CHUNK = 128


def _phase1_body(m, q_ref, k_ref, wq_ref, g_ref, cm_ref, cms):
    i = pl.program_id(0)
    qb = q_ref[...]
    ss = jnp.sum(qb * qb, axis=1, keepdims=True)
    qn = qb * jax.lax.rsqrt(ss + 1e-24)
    scT = jax.lax.dot_general(
        k_ref[...], qn.astype(jnp.bfloat16),
        (((1,), (1,)), ((), ())), preferred_element_type=jnp.float32)
    ri = jax.lax.broadcasted_iota(jnp.int32, (_MP, 1), 0)
    offv = jnp.where(ri < m, jnp.float32(32.0), -jnp.inf)
    invv = jnp.where(ri < m, (_MP - 1) - ri, 0)
    sc2 = scT + offv
    kb = jax.lax.bitcast_convert_type(sc2, jnp.int32)
    keyf = jax.lax.bitcast_convert_type(
        (kb & jnp.int32(-1024)) | invv, jnp.float32)
    gk = jnp.max(keyf, axis=0, keepdims=True)
    colp = jnp.max(sc2, axis=1, keepdims=True)
    gb = jax.lax.bitcast_convert_type(gk, jnp.int32)
    g = (_MP - 1) - (gb & 1023)
    s2 = jax.lax.bitcast_convert_type(gb & jnp.int32(-1024), jnp.float32)
    es = jnp.exp(s2)
    esT = jax.lax.transpose(jnp.broadcast_to(es, (128, _BLK)), (1, 0))
    wq_ref[...] = qn * esT
    g_ref[...] = g.reshape(1, 1, _BLK)

    @pl.when(i == 0)
    def _():
        cms[...] = colp

    @pl.when(i > 0)
    def _():
        cms[...] = jnp.maximum(cms[...], colp)

    @pl.when(i == pl.num_programs(0) - 1)
    def _():
        cm_ref[...] = cms[...]


def _scatter_body(n_tok, wq_hbm, g2_hbm, z_hbm, out_hbm,
                  idx_v, bufs, gsems, acc_sh):
    c = lax.axis_index("c")
    s = lax.axis_index("s")
    wid = s * _NC + c
    per_w = n_tok // (_NC * _NS)
    nch = per_w // _CHUNK

    @pl.when(s == 0)
    def _():
        pltpu.sync_copy(z_hbm, acc_sh)

    pltpu.sync_copy(g2_hbm.at[pl.ds(wid * nch, nch)], idx_v)

    base = wid * per_w
    gh = [None] * nch
    gh[0] = pltpu.async_copy(
        wq_hbm.at[pl.ds(base, _CHUNK)], bufs[0], gsems[0])
    plsc.subcore_barrier()
    for ch in range(nch):
        if ch + 1 < nch:
            gh[ch + 1] = pltpu.async_copy(
                wq_hbm.at[pl.ds(base + (ch + 1) * _CHUNK, _CHUNK)],
                bufs[ch + 1], gsems[ch + 1])
        gh[ch].wait()
        pltpu.sync_copy(bufs[ch], acc_sh.at[idx_v.at[ch]], add=True)

    plsc.subcore_barrier()
    rows_per_tile = _MP // _NS
    pltpu.sync_copy(acc_sh.at[pl.ds(s * rows_per_tile, rows_per_tile)],
                    out_hbm.at[pl.ds(c * _MP + s * rows_per_tile, rows_per_tile)])


def _finalize_body(m, np_, k_ref, *refs):
    acc_refs = refs[:np_]
    cm_refs = refs[np_:2 * np_]
    out_ref = refs[2 * np_]
    upd = acc_refs[0][0:_MP, :] + acc_refs[0][_MP:2 * _MP, :]
    cm = cm_refs[0][...]
    for j in range(1, np_):
        upd = upd + acc_refs[j][0:_MP, :] + acc_refs[j][_MP:2 * _MP, :]
        cm = jnp.maximum(cm, cm_refs[j][...])
    x = 1e-5 * (upd[0:m, :] * jnp.exp(-cm[0:m, :])) + k_ref[...]
    nrm = jnp.sqrt(jnp.sum(x * x, axis=1, keepdims=True))
    out_ref[...] = x / jnp.maximum(nrm, 1e-12)


_NP = 2


def kernel(query, keys):
    bs, t, d = query.shape
    m = keys.shape[0]
    n_tok = bs * t
    nb = n_tok // _BLK
    q2 = query.reshape(n_tok, d)
    kp = jnp.pad(keys, ((0, _MP - m), (0, 0))).astype(jnp.bfloat16)

    npb = nb // _NP
    ck_tok = n_tok // _NP
    nch = ck_tok // (_NC * _NS) // _CHUNK
    mesh = plsc.VectorSubcoreMesh(
        core_axis_name="c", subcore_axis_name="s",
        num_cores=_NC, num_subcores=_NS)
    zeros = jnp.zeros((_MP, d), jnp.float32)

    accs, cms = [], []
    for j in range(_NP):
        wq_j, g_j, cm_j = pl.pallas_call(
            functools.partial(_phase1_body, m),
            grid=(npb,),
            in_specs=[
                pl.BlockSpec((_BLK, d), lambda i, j=j: (j * npb + i, 0)),
                pl.BlockSpec((_MP, d), lambda i: (0, 0)),
            ],
            out_specs=[
                pl.BlockSpec((_BLK, d), lambda i: (i, 0)),
                pl.BlockSpec((1, 1, _BLK), lambda i: (i, 0, 0)),
                pl.BlockSpec((_MP, 1), lambda i: (0, 0)),
            ],
            out_shape=[
                jax.ShapeDtypeStruct((ck_tok, d), jnp.float32),
                jax.ShapeDtypeStruct((npb, 1, _BLK), jnp.int32),
                jax.ShapeDtypeStruct((_MP, 1), jnp.float32),
            ],
            scratch_shapes=[pltpu.VMEM((_MP, 1), jnp.float32)],
        )(q2, kp)

        acc_j = pl.kernel(
            functools.partial(_scatter_body, ck_tok),
            out_type=jax.ShapeDtypeStruct((_NC * _MP, d), jnp.float32),
            mesh=mesh,
            scratch_types=[
                pltpu.VMEM((nch, _CHUNK), jnp.int32),
                tuple(pltpu.VMEM((_CHUNK, d), jnp.float32) for _ in range(nch)),
                tuple(pltpu.SemaphoreType.DMA for _ in range(nch)),
                pltpu.VMEM_SHARED((_MP, d), jnp.float32),
            ],
        )(wq_j, g_j.reshape(ck_tok // _CHUNK, _CHUNK), zeros)
        accs.append(acc_j)
        cms.append(cm_j)

    out = pl.pallas_call(
        functools.partial(_finalize_body, m, _NP),
        in_specs=[pl.BlockSpec((m, d), lambda: (0, 0))]
        + [pl.BlockSpec((_NC * _MP, d), lambda: (0, 0))] * _NP
        + [pl.BlockSpec((_MP, 1), lambda: (0, 0))] * _NP,
        out_specs=pl.BlockSpec((m, d), lambda: (0, 0)),
        out_shape=jax.ShapeDtypeStruct((m, d), jnp.float32),
    )(keys, *accs, *cms)
    return out
